# baseline (speedup 1.0000x reference)
import jax
import jax.numpy as jnp
from jax import lax
from jax.experimental import pallas as pl
from jax.experimental.pallas import tpu as pltpu


def _pooler_body(x_ref, w_ref, b_ref, o_ref):
    xb = x_ref[:, 0, :].astype(jnp.bfloat16)
    y = lax.dot_general(
        xb,
        w_ref[...],
        dimension_numbers=(((1,), (1,)), ((), ())),
        preferred_element_type=jnp.float32,
    )
    o_ref[...] = jnp.tanh(y + b_ref[...]).astype(o_ref.dtype)


def kernel(x, weight, bias, *, block_b=128):
    B, S, D = x.shape
    assert weight.shape == (D, D) and bias.shape == (D,)
    assert B % block_b == 0

    b2d = bias.reshape(1, D).astype(jnp.float32)
    grid = (B // block_b,)

    cost = pl.CostEstimate(
        flops=2 * B * D * D,
        transcendentals=B * D,
        bytes_accessed=(D * D * jnp.dtype(weight.dtype).itemsize
                        + B * D * jnp.dtype(x.dtype).itemsize
                        + D * 4
                        + B * D * jnp.dtype(x.dtype).itemsize),
    )

    return pl.pallas_call(
        _pooler_body,
        out_shape=jax.ShapeDtypeStruct((B, D), x.dtype),
        grid=grid,
        in_specs=[
            pl.BlockSpec((block_b, 8, D), lambda b: (b, 0, 0)),
            pl.BlockSpec((D, D), lambda b: (0, 0)),
            pl.BlockSpec((1, D), lambda b: (0, 0)),
        ],
        out_specs=pl.BlockSpec((block_b, D), lambda b: (b, 0)),
        compiler_params=pltpu.CompilerParams(
            dimension_semantics=("parallel",),
            vmem_limit_bytes=48 * 1024 * 1024,
        ),
        cost_estimate=cost,
    )(x, weight, b2d)


# speedup vs baseline: 1.7419x; 1.7419x over previous
import functools

import jax
import jax.numpy as jnp
from jax import lax
from jax.experimental import pallas as pl
from jax.experimental.pallas import tpu as pltpu


def _pooler_body(x_hbm, w_ref, b_ref, o_hbm, x_vmem, o_vmem, xsems, osems,
                 *, block_b):
    i = pl.program_id(0)
    half = block_b // 2

    x_cps = []
    for c in range(2):
        cp = pltpu.make_async_copy(
            x_hbm.at[pl.ds(i * block_b + c * half, half), 0, :],
            x_vmem.at[pl.ds(c * half, half), :],
            xsems.at[c])
        cp.start()
        x_cps.append(cp)

    o_cps = []
    for c in range(2):
        x_cps[c].wait()
        xb = x_vmem[pl.ds(c * half, half), :].astype(jnp.bfloat16)
        y = lax.dot_general(
            xb,
            w_ref[...],
            dimension_numbers=(((1,), (1,)), ((), ())),
            preferred_element_type=jnp.float32,
        )
        o_vmem[pl.ds(c * half, half), :] = jnp.tanh(y + b_ref[...])
        cp = pltpu.make_async_copy(
            o_vmem.at[pl.ds(c * half, half), :],
            o_hbm.at[pl.ds(i * block_b + c * half, half), :],
            osems.at[c])
        cp.start()
        o_cps.append(cp)
    for cp in o_cps:
        cp.wait()


def kernel(x, weight, bias, *, block_b=512):
    B, S, D = x.shape
    assert weight.shape == (D, D) and bias.shape == (D,)
    assert B % block_b == 0 and block_b % 2 == 0

    b2d = bias.reshape(1, D).astype(jnp.float32)
    grid = (B // block_b,)

    cost = pl.CostEstimate(
        flops=2 * B * D * D,
        transcendentals=B * D,
        bytes_accessed=(D * D * jnp.dtype(weight.dtype).itemsize
                        + B * D * jnp.dtype(x.dtype).itemsize
                        + D * 4
                        + B * D * jnp.dtype(x.dtype).itemsize),
    )

    return pl.pallas_call(
        functools.partial(_pooler_body, block_b=block_b),
        out_shape=jax.ShapeDtypeStruct((B, D), x.dtype),
        grid=grid,
        in_specs=[
            pl.BlockSpec(memory_space=pl.ANY),
            pl.BlockSpec((D, D), lambda b: (0, 0)),
            pl.BlockSpec((1, D), lambda b: (0, 0)),
        ],
        out_specs=pl.BlockSpec(memory_space=pl.ANY),
        scratch_shapes=[
            pltpu.VMEM((block_b, D), jnp.float32),
            pltpu.VMEM((block_b, D), jnp.float32),
            pltpu.SemaphoreType.DMA((2,)),
            pltpu.SemaphoreType.DMA((2,)),
        ],
        compiler_params=pltpu.CompilerParams(
            dimension_semantics=("parallel",),
            vmem_limit_bytes=48 * 1024 * 1024,
        ),
        cost_estimate=cost,
    )(x, weight, b2d)


# speedup vs baseline: 1.8742x; 1.0760x over previous
import functools

import jax
import jax.numpy as jnp
from jax import lax
from jax.experimental import pallas as pl
from jax.experimental.pallas import tpu as pltpu


def _pooler_body(x_hbm, w_ref, b_ref, o_ref, x_vmem, sem, *, block_b):
    i = pl.program_id(0)
    cp = pltpu.make_async_copy(
        x_hbm.at[pl.ds(i * block_b, block_b), 0, :], x_vmem, sem)
    cp.start()
    cp.wait()
    y = lax.dot_general(
        x_vmem[...],
        w_ref[...],
        dimension_numbers=(((1,), (1,)), ((), ())),
        preferred_element_type=jnp.float32,
    )
    y = y + b_ref[...]
    o_ref[...] = jnp.tanh(y).astype(o_ref.dtype)


def kernel(x, weight, bias, *, block_b=512):
    B, S, D = x.shape
    assert weight.shape == (D, D) and bias.shape == (D,)
    assert B % block_b == 0

    b2d = bias.reshape(1, D).astype(jnp.float32)
    grid = (B // block_b,)

    cost = pl.CostEstimate(
        flops=2 * B * D * D,
        transcendentals=B * D,
        bytes_accessed=(D * D * jnp.dtype(weight.dtype).itemsize
                        + B * D * jnp.dtype(x.dtype).itemsize
                        + D * 4
                        + B * D * jnp.dtype(x.dtype).itemsize),
    )

    return pl.pallas_call(
        functools.partial(_pooler_body, block_b=block_b),
        out_shape=jax.ShapeDtypeStruct((B, D), x.dtype),
        grid=grid,
        in_specs=[
            pl.BlockSpec(memory_space=pl.ANY),
            pl.BlockSpec((D, D), lambda b: (0, 0)),
            pl.BlockSpec((1, D), lambda b: (0, 0)),
        ],
        out_specs=pl.BlockSpec((block_b, D), lambda b: (b, 0)),
        scratch_shapes=[
            pltpu.VMEM((block_b, D), jnp.float32),
            pltpu.SemaphoreType.DMA,
        ],
        compiler_params=pltpu.CompilerParams(
            dimension_semantics=("parallel",),
            vmem_limit_bytes=48 * 1024 * 1024,
        ),
        cost_estimate=cost,
    )(x, weight, b2d)


# speedup vs baseline: 2.2761x; 1.2145x over previous
import functools

import jax
import jax.numpy as jnp
from jax import lax
from jax.experimental import pallas as pl
from jax.experimental.pallas import tpu as pltpu


def _pooler_body(x_hbm, w_ref, b_ref, o_hbm, x_vmem, o_vmem, xsems, osems,
                 *, block_b, nc):
    ch = block_b // nc

    x_cps = []
    for c in range(nc):
        cp = pltpu.make_async_copy(
            x_hbm.at[pl.ds(c * ch, ch), 0, :],
            x_vmem.at[pl.ds(c * ch, ch), :],
            xsems.at[c])
        cp.start()
        x_cps.append(cp)

    o_cps = []
    for c in range(nc):
        x_cps[c].wait()
        y = lax.dot_general(
            x_vmem[pl.ds(c * ch, ch), :],
            w_ref[...],
            dimension_numbers=(((1,), (1,)), ((), ())),
            preferred_element_type=jnp.float32,
        )
        o_vmem[pl.ds(c * ch, ch), :] = jnp.tanh(y + b_ref[...])
        cp = pltpu.make_async_copy(
            o_vmem.at[pl.ds(c * ch, ch), :],
            o_hbm.at[pl.ds(c * ch, ch), :],
            osems.at[c])
        cp.start()
        o_cps.append(cp)
    for cp in o_cps:
        cp.wait()


def kernel(x, weight, bias, *, nc=2):
    B, S, D = x.shape
    assert weight.shape == (D, D) and bias.shape == (D,)
    assert B % nc == 0

    b2d = bias.reshape(1, D).astype(jnp.float32)

    cost = pl.CostEstimate(
        flops=2 * B * D * D,
        transcendentals=B * D,
        bytes_accessed=(D * D * jnp.dtype(weight.dtype).itemsize
                        + B * D * jnp.dtype(x.dtype).itemsize
                        + D * 4
                        + B * D * jnp.dtype(x.dtype).itemsize),
    )

    return pl.pallas_call(
        functools.partial(_pooler_body, block_b=B, nc=nc),
        out_shape=jax.ShapeDtypeStruct((B, D), x.dtype),
        grid=(1,),
        in_specs=[
            pl.BlockSpec(memory_space=pl.ANY),
            pl.BlockSpec((D, D), lambda b: (0, 0)),
            pl.BlockSpec((1, D), lambda b: (0, 0)),
        ],
        out_specs=pl.BlockSpec(memory_space=pl.ANY),
        scratch_shapes=[
            pltpu.VMEM((B, D), jnp.float32),
            pltpu.VMEM((B, D), jnp.float32),
            pltpu.SemaphoreType.DMA((nc,)),
            pltpu.SemaphoreType.DMA((nc,)),
        ],
        compiler_params=pltpu.CompilerParams(
            dimension_semantics=("arbitrary",),
            vmem_limit_bytes=48 * 1024 * 1024,
        ),
        cost_estimate=cost,
    )(x, weight, b2d)
